# baseline (speedup 1.0000x reference)
import functools

import jax
import jax.numpy as jnp
from jax import lax
from jax.experimental import pallas as pl
from jax.experimental.pallas import tpu as pltpu
from jax.experimental.pallas import tpu_sc as plsc

D = 768
BATCH = 4
SEQ = 2048
B_TOTAL = BATCH * SEQ
L = 16
NC = 2
NS = 16
NW = NC * NS
B_PER_W = B_TOTAL // NW
W_PER_ROW = SEQ // B_PER_W
C = 32
NCHUNK = B_PER_W // C
NWBUF = 2
NPBUF = 3

_mesh = plsc.VectorSubcoreMesh(core_axis_name="c", subcore_axis_name="s")


@functools.partial(
    pl.kernel,
    mesh=_mesh,
    out_type=jax.ShapeDtypeStruct((BATCH, SEQ, D), jnp.float32),
    scratch_types=(
        [pltpu.VMEM((B_PER_W,), jnp.int32)] * 2
        + [pltpu.VMEM((C, D), jnp.float32)] * (NWBUF + NPBUF)
        + [pltpu.SemaphoreType.DMA] * (NWBUF + 2 * NPBUF)
    ),
)
def _embed_lookup(ids_hbm, pids_hbm, wtab_hbm, ptab_hbm, out_hbm,
                  widx, pidx, *bufs_and_sems):
    wbuf = bufs_and_sems[0:NWBUF]
    pbuf = bufs_and_sems[NWBUF:NWBUF + NPBUF]
    rest = bufs_and_sems[NWBUF + NPBUF:]
    wsem = rest[0:NWBUF]
    psem = rest[NWBUF:NWBUF + NPBUF]
    osem = rest[NWBUF + NPBUF:NWBUF + 2 * NPBUF]

    wid = lax.axis_index("s") * NC + lax.axis_index("c")
    row = wid // W_PER_ROW
    col = (wid % W_PER_ROW) * B_PER_W
    pltpu.sync_copy(ids_hbm.at[row, pl.ds(col, B_PER_W)], widx)
    pltpu.sync_copy(pids_hbm.at[row, pl.ds(col, B_PER_W)], pidx)

    def issue_word_gather(g):
        b = g % NWBUF
        return pltpu.async_copy(
            wtab_hbm.at[widx.at[pl.ds(g * C, C)]], wbuf[b], wsem[b])

    def issue_pos_gather(g):
        b = g % NPBUF
        return pltpu.async_copy(
            ptab_hbm.at[pidx.at[pl.ds(g * C, C)]], pbuf[b], psem[b])

    wd = [None] * NCHUNK
    pd = [None] * NCHUNK
    od = [None] * NCHUNK
    wd[0] = issue_word_gather(0)
    pd[0] = issue_pos_gather(0)

    for g in range(NCHUNK):
        bw = g % NWBUF
        bp = g % NPBUF
        wd[g].wait()
        pd[g].wait()
        if g + 1 < NCHUNK:
            if g >= 2:
                od[g - 2].wait()
            wd[g + 1] = issue_word_gather(g + 1)
            pd[g + 1] = issue_pos_gather(g + 1)

        UNROLL = 12
        NGRP = D // L // UNROLL

        def add_grp(i, carry, _wb=wbuf[bw], _pb=pbuf[bp]):
            r = i // NGRP
            j0 = (i % NGRP) * UNROLL
            for j in range(UNROLL):
                off = pl.ds((j0 + j) * L, L)
                _pb[r, off] = _wb[r, off] + _pb[r, off]
            return carry

        lax.fori_loop(0, C * NGRP, add_grp, 0)
        od[g] = pltpu.async_copy(
            pbuf[bp], out_hbm.at[row, pl.ds(col + g * C, C)], osem[bp])

    for g in range(NCHUNK - 3, NCHUNK):
        od[g].wait()


def kernel(input_ids, position_ids, attention_mask, word_embeddings,
           position_embeddings):
    return _embed_lookup(input_ids, position_ids, word_embeddings,
                         position_embeddings)


# speedup vs baseline: 1.4517x; 1.4517x over previous
import functools

import jax
import jax.numpy as jnp
from jax import lax
from jax.experimental import pallas as pl
from jax.experimental.pallas import tpu as pltpu
from jax.experimental.pallas import tpu_sc as plsc

D = 768
BATCH = 4
SEQ = 2048
B_TOTAL = BATCH * SEQ
L = 16
NC = 2
NS = 16
NW = NC * NS
B_PER_W = B_TOTAL // NW
W_PER_ROW = SEQ // B_PER_W
C = 32
NCHUNK = B_PER_W // C
NWBUF = 2
NPBUF = 3

_mesh = plsc.VectorSubcoreMesh(core_axis_name="c", subcore_axis_name="s")


@functools.partial(
    pl.kernel,
    mesh=_mesh,
    out_type=jax.ShapeDtypeStruct((BATCH, SEQ, D), jnp.float32),
    scratch_types=(
        [pltpu.VMEM((B_PER_W,), jnp.int32)] * 2
        + [pltpu.VMEM((C, D), jnp.float32)] * (NWBUF + NPBUF)
        + [pltpu.SemaphoreType.DMA] * (NWBUF + 2 * NPBUF)
    ),
)
def _embed_lookup(ids_hbm, pids_hbm, wtab_hbm, ptab_hbm, out_hbm,
                  widx, pidx, *bufs_and_sems):
    wbuf = bufs_and_sems[0:NWBUF]
    pbuf = bufs_and_sems[NWBUF:NWBUF + NPBUF]
    rest = bufs_and_sems[NWBUF + NPBUF:]
    wsem = rest[0:NWBUF]
    psem = rest[NWBUF:NWBUF + NPBUF]
    osem = rest[NWBUF + NPBUF:NWBUF + 2 * NPBUF]

    wid = lax.axis_index("s") * NC + lax.axis_index("c")
    row = wid // W_PER_ROW
    col = (wid % W_PER_ROW) * B_PER_W
    pltpu.sync_copy(ids_hbm.at[row, pl.ds(col, B_PER_W)], widx)
    pltpu.sync_copy(pids_hbm.at[row, pl.ds(col, B_PER_W)], pidx)

    def issue_word_gather(g):
        b = g % NWBUF
        return pltpu.async_copy(
            wtab_hbm.at[widx.at[pl.ds(g * C, C)]], wbuf[b], wsem[b])

    def issue_pos_gather(g):
        b = g % NPBUF
        return pltpu.async_copy(
            ptab_hbm.at[pidx.at[pl.ds(g * C, C)]], pbuf[b], psem[b])

    wd = [None] * NCHUNK
    pd = [None] * NCHUNK
    od = [None] * NCHUNK
    wd[0] = issue_word_gather(0)
    pd[0] = issue_pos_gather(0)

    for g in range(NCHUNK):
        bw = g % NWBUF
        bp = g % NPBUF
        wd[g].wait()
        pd[g].wait()
        if g + 1 < NCHUNK:
            if g >= 2:
                od[g - 2].wait()
            wd[g + 1] = issue_word_gather(g + 1)
            pd[g + 1] = issue_pos_gather(g + 1)

        def add_row(r, carry, _wb=wbuf[bw], _pb=pbuf[bp]):
            for j in range(D // L):
                s = _wb[r, pl.ds(j * L, L)] + _pb[r, pl.ds(j * L, L)]
                _pb[r, pl.ds(j * L, L)] = s
            return carry

        lax.fori_loop(0, C, add_row, 0)
        od[g] = pltpu.async_copy(
            pbuf[bp], out_hbm.at[row, pl.ds(col + g * C, C)], osem[bp])

    for g in range(NCHUNK - 3, NCHUNK):
        od[g].wait()


def kernel(input_ids, position_ids, attention_mask, word_embeddings,
           position_embeddings):
    return _embed_lookup(input_ids, position_ids, word_embeddings,
                         position_embeddings)
